# baseline (speedup 1.0000x reference)
import functools

import jax
import jax.numpy as jnp
from jax import lax
from jax.experimental import pallas as pl
from jax.experimental.pallas import tpu as pltpu
from jax.experimental.pallas import tpu_sc as plsc

NN = 10000
EE = 320000
DD = 128
NPAD = 10240
NC = 2
NS = 16
LL = 16
NW = NC * NS
EPW = EE // NW
CH = 80
NCHUNK = EPW // CH
CSR_C = 400
NCSR = EPW // CSR_C
NCHK = NPAD // NS
SENT = -3.0e38

_mesh = plsc.VectorSubcoreMesh(core_axis_name="c", subcore_axis_name="s")
_SC_PARAMS = pltpu.CompilerParams(needs_layout_passes=False)
_SC_PARAMS_A = pltpu.CompilerParams(
    needs_layout_passes=False, use_tc_tiling_on_sc=False)


def _vsqrt(x):
    xi = plsc.bitcast(x, jnp.int32)
    yi = jnp.int32(0x5F3759DF) - (xi >> 1)
    y = plsc.bitcast(yi, jnp.float32)
    for _ in range(4):
        y = y * (1.5 - 0.5 * x * y * y)
    return jnp.where(x <= 0.0, 0.0, x * y)


def _worker_id():
    c = lax.axis_index("c")
    s = lax.axis_index("s")
    return c, s, c * NS + s


@functools.partial(
    pl.kernel,
    out_type=jax.ShapeDtypeStruct((NC, 4, NPAD), jnp.float32),
    mesh=_mesh,
    compiler_params=_SC_PARAMS_A,
    scratch_types=[
        pltpu.VMEM((NPAD,), jnp.float32),
        pltpu.VMEM((NPAD,), jnp.float32),
        pltpu.VMEM((NPAD,), jnp.float32),
        pltpu.VMEM((NPAD,), jnp.float32),
        pltpu.VMEM((NPAD,), jnp.float32),
        pltpu.VMEM((EPW,), jnp.int32),
        pltpu.VMEM((EPW,), jnp.int32),
        pltpu.VMEM((2, CH, DD // 2), jnp.int32),
        pltpu.VMEM((2, CH, DD // 2), jnp.int32),
        pltpu.VMEM((LL * LL,), jnp.float32),
        pltpu.VMEM((2, CSR_C), jnp.int32),
        pltpu.VMEM((2, CSR_C), jnp.int32),
        pltpu.VMEM((2, CSR_C), jnp.float32),
        pltpu.VMEM((NCHK,), jnp.float32),
        pltpu.VMEM((NCHK,), jnp.float32),
        pltpu.VMEM_SHARED((NS, 2, NPAD), jnp.float32),
        pltpu.SemaphoreType.DMA((2,)),
        pltpu.SemaphoreType.DMA((2,)),
        pltpu.SemaphoreType.DMA((2,)),
    ],
)
def _stage_a(src_h, dst_h, col_h, row_h, val_h, pred_h, feats_h, out_h,
             predv, ngacc, cntacc, divacc, rracc, srcb, dstb, rs, rd,
             tbuf, colb, rowb, valb, tmp, mrg, shared, sems, semd, semc):
    cid, sid, wid = _worker_id()
    base = wid * EPW

    pltpu.sync_copy(pred_h, predv)
    pltpu.sync_copy(src_h.at[pl.ds(base, EPW)], srcb)
    pltpu.sync_copy(dst_h.at[pl.ds(base, EPW)], dstb)

    def zbody(k, _):
        z = jnp.zeros((LL,), jnp.float32)
        ngacc[pl.ds(k * LL, LL)] = z
        cntacc[pl.ds(k * LL, LL)] = z
        divacc[pl.ds(k * LL, LL)] = z
        rracc[pl.ds(k * LL, LL)] = z
        return 0
    lax.fori_loop(0, NPAD // LL, zbody, 0)

    ones16 = jnp.ones((LL,), jnp.float32)

    def launch(i):
        slot = i & 1
        pltpu.async_copy(
            feats_h.at[srcb.at[pl.ds(i * CH, CH)]], rs.at[slot],
            sems.at[slot])
        pltpu.async_copy(
            feats_h.at[dstb.at[pl.ds(i * CH, CH)]], rd.at[slot],
            semd.at[slot])

    def do_chunk(i):
        slot = i & 1
        pltpu.make_async_copy(
            feats_h.at[srcb.at[pl.ds(i * CH, CH)]], rs.at[slot],
            sems.at[slot]).wait()
        pltpu.make_async_copy(
            feats_h.at[dstb.at[pl.ds(i * CH, CH)]], rd.at[slot],
            semd.at[slot]).wait()

        @pl.when(i + 1 < NCHUNK)
        def _():
            launch(i + 1)

        for g in range(CH // LL):
            s16 = srcb[pl.ds(i * CH + g * LL, LL)]
            d16 = dstb[pl.ds(i * CH + g * LL, LL)]

            himask = jnp.int32(-65536)

            for el in range(LL):
                e = g * LL + el
                acc = jnp.zeros((LL,), jnp.float32)
                for k in range(DD // (2 * LL)):
                    a = rs[slot, e, pl.ds(k * LL, LL)]
                    b2 = rd[slot, e, pl.ds(k * LL, LL)]
                    dlo = (plsc.bitcast(a << 16, jnp.float32)
                           - plsc.bitcast(b2 << 16, jnp.float32))
                    dhi = (plsc.bitcast(a & himask, jnp.float32)
                           - plsc.bitcast(b2 & himask, jnp.float32))
                    acc = acc + dlo * dlo + dhi * dhi
                tbuf[pl.ds(el * LL, LL)] = acc

            d2 = jnp.zeros((LL,), jnp.float32)
            rowi = lax.iota(jnp.int32, LL) * LL
            for c in range(LL):
                d2 = d2 + plsc.load_gather(tbuf, [rowi + c])

            ps = plsc.load_gather(predv, [s16])
            pd = plsc.load_gather(predv, [d16])
            pdiff = jnp.abs(pd - ps)
            flux = ps - pd
            fd = _vsqrt(d2)
            gval = pdiff + 0.1 * fd
            plsc.addupdate_scatter(ngacc, [d16], gval)
            plsc.addupdate_scatter(cntacc, [d16], ones16)
            plsc.addupdate_scatter(divacc, [d16], flux)

    launch(0)

    def chunk_body(i, _):
        do_chunk(i)
        return 0
    lax.fori_loop(0, NCHUNK, chunk_body, 0)

    def csr_launch(t, slot):
        off = base + t * CSR_C
        pltpu.async_copy(col_h.at[pl.ds(off, CSR_C)], colb.at[slot],
                         semc.at[slot])
        pltpu.async_copy(row_h.at[pl.ds(off, CSR_C)], rowb.at[slot],
                         semc.at[slot])
        pltpu.async_copy(val_h.at[pl.ds(off, CSR_C)], valb.at[slot],
                         semc.at[slot])

    def csr_chunk(t, slot):
        off = base + t * CSR_C
        pltpu.make_async_copy(col_h.at[pl.ds(off, CSR_C)], colb.at[slot],
                              semc.at[slot]).wait()
        pltpu.make_async_copy(row_h.at[pl.ds(off, CSR_C)], rowb.at[slot],
                              semc.at[slot]).wait()
        pltpu.make_async_copy(val_h.at[pl.ds(off, CSR_C)], valb.at[slot],
                              semc.at[slot]).wait()

        @pl.when(t + 1 < NCSR)
        def _():
            csr_launch(t + 1, 1 - slot)

        def gbody(gg, _):
            c16 = colb[slot, pl.ds(gg * LL, LL)]
            r16 = rowb[slot, pl.ds(gg * LL, LL)]
            v16 = valb[slot, pl.ds(gg * LL, LL)]
            pv = plsc.load_gather(predv, [c16])
            plsc.addupdate_scatter(rracc, [r16], v16 * pv)
            return 0
        lax.fori_loop(0, CSR_C // LL, gbody, 0)

    csr_launch(0, 0)

    def csr_pair(j, _):
        csr_chunk(2 * j, 0)
        csr_chunk(2 * j + 1, 1)
        return 0
    lax.fori_loop(0, (NCSR - 1) // 2, csr_pair, 0)
    csr_chunk(NCSR - 1, 0)

    for half, pair in enumerate(((ngacc, cntacc), (divacc, rracc))):
        if half:
            plsc.subcore_barrier()
        pltpu.sync_copy(pair[0], shared.at[sid, 0])
        pltpu.sync_copy(pair[1], shared.at[sid, 1])
        plsc.subcore_barrier()

        for r in range(2):
            def zb(k, _):
                mrg[pl.ds(k * LL, LL)] = jnp.zeros((LL,), jnp.float32)
                return 0
            lax.fori_loop(0, NCHK // LL, zb, 0)

            def slot_body(t, _):
                pltpu.sync_copy(
                    shared.at[t, r, pl.ds(sid * NCHK, NCHK)], tmp)

                def addk(k, _):
                    sl = pl.ds(k * LL, LL)
                    mrg[sl] = mrg[sl] + tmp[sl]
                    return 0
                lax.fori_loop(0, NCHK // LL, addk, 0)
                return 0
            lax.fori_loop(0, NS, slot_body, 0)
            pltpu.sync_copy(
                mrg, out_h.at[cid, half * 2 + r, pl.ds(sid * NCHK, NCHK)])


def _make_hop(with_prev, decay):
    scratch = [
        pltpu.VMEM((NPAD,), jnp.float32),
        pltpu.VMEM((NPAD,), jnp.float32),
        pltpu.VMEM((EPW,), jnp.int32),
        pltpu.VMEM((EPW,), jnp.int32),
        pltpu.VMEM((NCHK,), jnp.float32),
        pltpu.VMEM((NCHK,), jnp.float32),
        pltpu.VMEM((NPAD,), jnp.float32),
        pltpu.VMEM((NPAD,), jnp.float32),
        pltpu.VMEM_SHARED((NS, NPAD), jnp.float32),
    ]

    def body(*refs):
        if with_prev:
            (w_h, mp_h, src_h, dst_h, out_h,
             wv, macc, srcb, dstb, tmp, mrg, mp0, mp1, shared) = refs
        else:
            (p_h, src_h, dst_h, out_h, w_out,
             wv, macc, srcb, dstb, tmp, mrg, mp0, mp1, shared) = refs
        cid, sid, wid = _worker_id()
        base = wid * EPW

        if with_prev:
            pltpu.sync_copy(w_h, wv)
            pltpu.sync_copy(mp_h.at[0], mp0)
            pltpu.sync_copy(mp_h.at[1], mp1)

            def upd(k, _):
                sl = pl.ds(k * LL, LL)
                m = jnp.maximum(mp0[sl], mp1[sl])
                fx = jnp.where(m > -1.0e38, m, 0.0)
                wv[sl] = jnp.maximum(wv[sl], decay * fx)
                return 0
            lax.fori_loop(0, NPAD // LL, upd, 0)
        else:
            pltpu.sync_copy(p_h.at[0], wv)
            pltpu.sync_copy(p_h.at[4], mp0)
            pltpu.sync_copy(p_h.at[1], mp1)

            def u1(k, _):
                sl = pl.ds(k * LL, LL)
                wv[sl] = wv[sl] + mp0[sl]
                return 0
            lax.fori_loop(0, NPAD // LL, u1, 0)
            pltpu.sync_copy(p_h.at[5], mp0)

            def u2(k, _):
                sl = pl.ds(k * LL, LL)
                wv[sl] = wv[sl] / (mp0[sl] + mp1[sl] + 1.0)
                return 0
            lax.fori_loop(0, NPAD // LL, u2, 0)

            def u3(k, acc):
                return acc + wv[pl.ds(k * LL, LL)]
            vec = lax.fori_loop(0, NPAD // LL, u3,
                                jnp.zeros((LL,), jnp.float32))
            mean = jnp.sum(vec) * jnp.float32(1.0 / NN)

            def u4(k, _):
                sl = pl.ds(k * LL, LL)
                wv[sl] = 1.0 + wv[sl] / (mean + 1e-8)
                return 0
            lax.fori_loop(0, NPAD // LL, u4, 0)

            @pl.when(cid == 0)
            def _():
                pltpu.sync_copy(wv.at[pl.ds(sid * NCHK, NCHK)],
                                w_out.at[pl.ds(sid * NCHK, NCHK)])

        def ini(k, _):
            macc[pl.ds(k * LL, LL)] = jnp.full((LL,), SENT, jnp.float32)
            return 0
        lax.fori_loop(0, NPAD // LL, ini, 0)

        pltpu.sync_copy(src_h.at[pl.ds(base, EPW)], srcb)
        pltpu.sync_copy(dst_h.at[pl.ds(base, EPW)], dstb)

        def gbody(gg, _):
            s16 = srcb[pl.ds(gg * LL, LL)]
            d16 = dstb[pl.ds(gg * LL, LL)]
            val = plsc.load_gather(wv, [s16])
            cur = plsc.load_gather(macc, [d16])
            need = val > cur

            def cond(carry):
                return carry[0]

            def wbody(carry):
                _, nd = carry
                plsc.store_scatter(macc, [d16], val, mask=nd)
                cur2 = plsc.load_gather(macc, [d16])
                nd2 = val > cur2
                return (jnp.any(nd2), nd2)

            lax.while_loop(cond, wbody, (jnp.any(need), need))
            return 0
        lax.fori_loop(0, EPW // LL, gbody, 0)

        pltpu.sync_copy(macc, shared.at[sid])
        plsc.subcore_barrier()

        def zb(k, _):
            mrg[pl.ds(k * LL, LL)] = jnp.full((LL,), SENT, jnp.float32)
            return 0
        lax.fori_loop(0, NCHK // LL, zb, 0)

        def slot_body(t, _):
            pltpu.sync_copy(shared.at[t, pl.ds(sid * NCHK, NCHK)], tmp)

            def mk(k, _):
                sl = pl.ds(k * LL, LL)
                mrg[sl] = jnp.maximum(mrg[sl], tmp[sl])
                return 0
            lax.fori_loop(0, NCHK // LL, mk, 0)
            return 0
        lax.fori_loop(0, NS, slot_body, 0)
        pltpu.sync_copy(mrg, out_h.at[cid, pl.ds(sid * NCHK, NCHK)])

    if with_prev:
        out_type = jax.ShapeDtypeStruct((NC, NPAD), jnp.float32)
    else:
        out_type = (jax.ShapeDtypeStruct((NC, NPAD), jnp.float32),
                    jax.ShapeDtypeStruct((NPAD,), jnp.float32))
    return functools.partial(
        pl.kernel,
        out_type=out_type,
        mesh=_mesh,
        compiler_params=_SC_PARAMS,
        scratch_types=scratch,
    )(body)


_hop1 = _make_hop(False, 0.5)
_hop2 = _make_hop(True, 0.5)


def _stage_d_body(w1_ref, m1_ref, m2_ref, p_ref, b_ref, pred_ref, mf_ref,
                  o_ref):
    def fx(m):
        return jnp.where(m > -1.0e38, m, 0.0)

    w1 = w1_ref[...]
    w2 = jnp.maximum(w1, 0.5 * fx(jnp.maximum(m1_ref[0], m1_ref[1])))
    w3 = jnp.maximum(w2, 0.25 * fx(jnp.maximum(m2_ref[0], m2_ref[1])))
    resid = (p_ref[3] + p_ref[7]) - b_ref[...]
    loss_pde = jnp.sum(w3 * resid * resid) / NN
    dv = p_ref[2] + p_ref[6]
    loss_cons = jnp.sum(dv * dv) / NN
    pred = pred_ref[...]

    def mmse(mf, t):
        c = jnp.maximum(jnp.sum(mf), 1.0)
        return jnp.sum(mf * (pred - t) ** 2) / c

    loss_bc = mmse(mf_ref[0], 0.1) + mmse(mf_ref[1], 0.0) + mmse(mf_ref[2], 0.0)
    o_ref[0] = loss_pde + 10.0 * loss_bc + loss_cons
    o_ref[1] = loss_pde
    o_ref[2] = loss_bc
    o_ref[3] = loss_cons


_stage_d = pl.pallas_call(
    _stage_d_body,
    out_shape=jax.ShapeDtypeStruct((4,), jnp.float32),
    out_specs=pl.BlockSpec(memory_space=pltpu.MemorySpace.SMEM),
)


def _pad1(x, val=0.0):
    return jnp.concatenate(
        [x, jnp.full((NPAD - NN,), val, x.dtype)])


def kernel(pred, feats, A_row_ptr, A_col_ind, A_vals, A_row_idx, b,
           edge_index, epoch, inlet_mask, outlet_mask, wall_mask):
    f32 = jnp.float32
    i32 = jnp.int32
    src = edge_index[0].astype(i32)
    dst = edge_index[1].astype(i32)
    col = A_col_ind.astype(i32)
    row = A_row_idx.astype(i32)
    vals = A_vals.astype(f32)
    pred_p = _pad1(pred.astype(f32))
    b_p = _pad1(b.astype(f32))

    feats_pk = lax.bitcast_convert_type(
        feats.astype(jnp.bfloat16).reshape(NN, DD // 2, 2), i32)
    partials = _stage_a(src, dst, col, row, vals, pred_p,
                        feats_pk)
    m1p, w1f = _hop1(partials.reshape(NC * 4, NPAD), src, dst)
    m2p = _hop2(w1f, m1p, src, dst)

    sh = (NPAD // 128, 128)
    mf = jnp.stack([
        _pad1(inlet_mask.astype(f32)).reshape(sh),
        _pad1(outlet_mask.astype(f32)).reshape(sh),
        _pad1(wall_mask.astype(f32)).reshape(sh),
    ])
    out4 = _stage_d(
        w1f.reshape(sh),
        m1p.reshape(2, *sh),
        m2p.reshape(2, *sh),
        partials.reshape(8, *sh),
        b_p.reshape(sh),
        pred_p.reshape(sh),
        mf,
    )
    return (out4[0], out4[1], out4[2], out4[3])


# speedup vs baseline: 1.1205x; 1.1205x over previous
import functools

import jax
import jax.numpy as jnp
from jax import lax
from jax.experimental import pallas as pl
from jax.experimental.pallas import tpu as pltpu
from jax.experimental.pallas import tpu_sc as plsc

NN = 10000
EE = 320000
DD = 128
NPAD = 10240
NC = 2
NS = 16
LL = 16
NW = NC * NS
EPW = EE // NW
CH = 80
NCHUNK = EPW // CH
CSR_C = 400
NCSR = EPW // CSR_C
NCHK = NPAD // NS
SENT = -3.0e38

_mesh = plsc.VectorSubcoreMesh(core_axis_name="c", subcore_axis_name="s")
_SC_PARAMS = pltpu.CompilerParams(needs_layout_passes=False)
_SC_PARAMS_A = pltpu.CompilerParams(
    needs_layout_passes=False, use_tc_tiling_on_sc=False)


def _vsqrt(x):
    xi = plsc.bitcast(x, jnp.int32)
    yi = jnp.int32(0x5F3759DF) - (xi >> 1)
    y = plsc.bitcast(yi, jnp.float32)
    for _ in range(4):
        y = y * (1.5 - 0.5 * x * y * y)
    return jnp.where(x <= 0.0, 0.0, x * y)


def _worker_id():
    c = lax.axis_index("c")
    s = lax.axis_index("s")
    return c, s, c * NS + s


@functools.partial(
    pl.kernel,
    out_type=jax.ShapeDtypeStruct((NC, 4, NPAD), jnp.float32),
    mesh=_mesh,
    compiler_params=_SC_PARAMS_A,
    scratch_types=[
        pltpu.VMEM((NPAD,), jnp.float32),
        pltpu.VMEM((NPAD,), jnp.float32),
        pltpu.VMEM((NPAD,), jnp.float32),
        pltpu.VMEM((NPAD,), jnp.float32),
        pltpu.VMEM((NPAD,), jnp.float32),
        pltpu.VMEM((EPW,), jnp.int32),
        pltpu.VMEM((EPW,), jnp.int32),
        pltpu.VMEM((2, CH, DD // 2), jnp.int32),
        pltpu.VMEM((2, CH, DD // 2), jnp.int32),
        pltpu.VMEM((LL * LL,), jnp.float32),
        pltpu.VMEM((2, CSR_C), jnp.int32),
        pltpu.VMEM((2, CSR_C), jnp.int32),
        pltpu.VMEM((2, CSR_C), jnp.float32),
        pltpu.VMEM((NCHK,), jnp.float32),
        pltpu.VMEM((NCHK,), jnp.float32),
        pltpu.VMEM_SHARED((NS, 2, NPAD), jnp.float32),
        pltpu.SemaphoreType.DMA((2,)),
        pltpu.SemaphoreType.DMA((2,)),
        pltpu.SemaphoreType.DMA((2,)),
    ],
)
def _stage_a(src_h, dst_h, col_h, row_h, val_h, pred_h, feats_h, out_h,
             predv, ngacc, cntacc, divacc, rracc, srcb, dstb, rs, rd,
             tbuf, colb, rowb, valb, tmp, mrg, shared, sems, semd, semc):
    cid, sid, wid = _worker_id()
    base = wid * EPW

    pltpu.sync_copy(pred_h, predv)
    pltpu.sync_copy(src_h.at[pl.ds(base, EPW)], srcb)
    pltpu.sync_copy(dst_h.at[pl.ds(base, EPW)], dstb)

    def zbody(k, _):
        z = jnp.zeros((LL,), jnp.float32)
        ngacc[pl.ds(k * LL, LL)] = z
        cntacc[pl.ds(k * LL, LL)] = z
        divacc[pl.ds(k * LL, LL)] = z
        rracc[pl.ds(k * LL, LL)] = z
        return 0
    lax.fori_loop(0, NPAD // LL, zbody, 0)

    ones16 = jnp.ones((LL,), jnp.float32)

    def launch(i, slot):
        pltpu.async_copy(
            feats_h.at[srcb.at[pl.ds(i * CH, CH)]], rs.at[slot],
            sems.at[slot])
        pltpu.async_copy(
            feats_h.at[dstb.at[pl.ds(i * CH, CH)]], rd.at[slot],
            semd.at[slot])

    def do_chunk(i, slot):
        pltpu.make_async_copy(
            feats_h.at[srcb.at[pl.ds(i * CH, CH)]], rs.at[slot],
            sems.at[slot]).wait()
        pltpu.make_async_copy(
            feats_h.at[dstb.at[pl.ds(i * CH, CH)]], rd.at[slot],
            semd.at[slot]).wait()

        @pl.when(i + 1 < NCHUNK)
        def _():
            launch(i + 1, 1 - slot)

        for g in range(CH // LL):
            s16 = srcb[pl.ds(i * CH + g * LL, LL)]
            d16 = dstb[pl.ds(i * CH + g * LL, LL)]

            himask = jnp.int32(-65536)

            def erow4(q, _):
                for j in range(4):
                    e = g * LL + q * 4 + j
                    acc = jnp.zeros((LL,), jnp.float32)
                    for k in range(DD // (2 * LL)):
                        a = rs[slot, e, pl.ds(k * LL, LL)]
                        b2 = rd[slot, e, pl.ds(k * LL, LL)]
                        dlo = (plsc.bitcast(a << 16, jnp.float32)
                               - plsc.bitcast(b2 << 16, jnp.float32))
                        dhi = (plsc.bitcast(a & himask, jnp.float32)
                               - plsc.bitcast(b2 & himask, jnp.float32))
                        acc = acc + dlo * dlo + dhi * dhi
                    tbuf[pl.ds((q * 4 + j) * LL, LL)] = acc
                return 0
            lax.fori_loop(0, 4, erow4, 0)

            d2 = jnp.zeros((LL,), jnp.float32)
            rowi = lax.iota(jnp.int32, LL) * LL
            for c in range(LL):
                d2 = d2 + plsc.load_gather(tbuf, [rowi + c])

            ps = plsc.load_gather(predv, [s16])
            pd = plsc.load_gather(predv, [d16])
            pdiff = jnp.abs(pd - ps)
            flux = ps - pd
            fd = _vsqrt(d2)
            gval = pdiff + 0.1 * fd
            plsc.addupdate_scatter(ngacc, [d16], gval)
            plsc.addupdate_scatter(cntacc, [d16], ones16)
            plsc.addupdate_scatter(divacc, [d16], flux)

    launch(0, 0)

    def pair_body(j, _):
        do_chunk(2 * j, 0)
        do_chunk(2 * j + 1, 1)
        return 0
    lax.fori_loop(0, (NCHUNK - 1) // 2, pair_body, 0)
    do_chunk(NCHUNK - 1, 0)

    def csr_launch(t, slot):
        off = base + t * CSR_C
        pltpu.async_copy(col_h.at[pl.ds(off, CSR_C)], colb.at[slot],
                         semc.at[slot])
        pltpu.async_copy(row_h.at[pl.ds(off, CSR_C)], rowb.at[slot],
                         semc.at[slot])
        pltpu.async_copy(val_h.at[pl.ds(off, CSR_C)], valb.at[slot],
                         semc.at[slot])

    def csr_chunk(t, slot):
        off = base + t * CSR_C
        pltpu.make_async_copy(col_h.at[pl.ds(off, CSR_C)], colb.at[slot],
                              semc.at[slot]).wait()
        pltpu.make_async_copy(row_h.at[pl.ds(off, CSR_C)], rowb.at[slot],
                              semc.at[slot]).wait()
        pltpu.make_async_copy(val_h.at[pl.ds(off, CSR_C)], valb.at[slot],
                              semc.at[slot]).wait()

        @pl.when(t + 1 < NCSR)
        def _():
            csr_launch(t + 1, 1 - slot)

        def gbody(gg, _):
            c16 = colb[slot, pl.ds(gg * LL, LL)]
            r16 = rowb[slot, pl.ds(gg * LL, LL)]
            v16 = valb[slot, pl.ds(gg * LL, LL)]
            pv = plsc.load_gather(predv, [c16])
            plsc.addupdate_scatter(rracc, [r16], v16 * pv)
            return 0
        lax.fori_loop(0, CSR_C // LL, gbody, 0)

    csr_launch(0, 0)

    def csr_pair(j, _):
        csr_chunk(2 * j, 0)
        csr_chunk(2 * j + 1, 1)
        return 0
    lax.fori_loop(0, (NCSR - 1) // 2, csr_pair, 0)
    csr_chunk(NCSR - 1, 0)

    for half, pair in enumerate(((ngacc, cntacc), (divacc, rracc))):
        if half:
            plsc.subcore_barrier()
        pltpu.sync_copy(pair[0], shared.at[sid, 0])
        pltpu.sync_copy(pair[1], shared.at[sid, 1])
        plsc.subcore_barrier()

        for r in range(2):
            def zb(k, _):
                mrg[pl.ds(k * LL, LL)] = jnp.zeros((LL,), jnp.float32)
                return 0
            lax.fori_loop(0, NCHK // LL, zb, 0)

            def slot_body(t, _):
                pltpu.sync_copy(
                    shared.at[t, r, pl.ds(sid * NCHK, NCHK)], tmp)

                def addk(k, _):
                    sl = pl.ds(k * LL, LL)
                    mrg[sl] = mrg[sl] + tmp[sl]
                    return 0
                lax.fori_loop(0, NCHK // LL, addk, 0)
                return 0
            lax.fori_loop(0, NS, slot_body, 0)
            pltpu.sync_copy(
                mrg, out_h.at[cid, half * 2 + r, pl.ds(sid * NCHK, NCHK)])


def _make_hop(with_prev, decay):
    scratch = [
        pltpu.VMEM((NPAD,), jnp.float32),
        pltpu.VMEM((NPAD,), jnp.float32),
        pltpu.VMEM((EPW,), jnp.int32),
        pltpu.VMEM((EPW,), jnp.int32),
        pltpu.VMEM((NCHK,), jnp.float32),
        pltpu.VMEM((NCHK,), jnp.float32),
        pltpu.VMEM((NPAD,), jnp.float32),
        pltpu.VMEM((NPAD,), jnp.float32),
        pltpu.VMEM_SHARED((NS, NPAD), jnp.float32),
    ]

    def body(*refs):
        if with_prev:
            (w_h, mp_h, src_h, dst_h, out_h,
             wv, macc, srcb, dstb, tmp, mrg, mp0, mp1, shared) = refs
        else:
            (p_h, src_h, dst_h, out_h, w_out,
             wv, macc, srcb, dstb, tmp, mrg, mp0, mp1, shared) = refs
        cid, sid, wid = _worker_id()
        base = wid * EPW

        if with_prev:
            pltpu.sync_copy(w_h, wv)
            pltpu.sync_copy(mp_h.at[0], mp0)
            pltpu.sync_copy(mp_h.at[1], mp1)

            def upd(k, _):
                sl = pl.ds(k * LL, LL)
                m = jnp.maximum(mp0[sl], mp1[sl])
                fx = jnp.where(m > -1.0e38, m, 0.0)
                wv[sl] = jnp.maximum(wv[sl], decay * fx)
                return 0
            lax.fori_loop(0, NPAD // LL, upd, 0)
        else:
            pltpu.sync_copy(p_h.at[0], wv)
            pltpu.sync_copy(p_h.at[4], mp0)
            pltpu.sync_copy(p_h.at[1], mp1)

            def u1(k, _):
                sl = pl.ds(k * LL, LL)
                wv[sl] = wv[sl] + mp0[sl]
                return 0
            lax.fori_loop(0, NPAD // LL, u1, 0)
            pltpu.sync_copy(p_h.at[5], mp0)

            def u2(k, _):
                sl = pl.ds(k * LL, LL)
                wv[sl] = wv[sl] / (mp0[sl] + mp1[sl] + 1.0)
                return 0
            lax.fori_loop(0, NPAD // LL, u2, 0)

            def u3(k, acc):
                return acc + wv[pl.ds(k * LL, LL)]
            vec = lax.fori_loop(0, NPAD // LL, u3,
                                jnp.zeros((LL,), jnp.float32))
            mean = jnp.sum(vec) * jnp.float32(1.0 / NN)

            def u4(k, _):
                sl = pl.ds(k * LL, LL)
                wv[sl] = 1.0 + wv[sl] / (mean + 1e-8)
                return 0
            lax.fori_loop(0, NPAD // LL, u4, 0)

            @pl.when(cid == 0)
            def _():
                pltpu.sync_copy(wv.at[pl.ds(sid * NCHK, NCHK)],
                                w_out.at[pl.ds(sid * NCHK, NCHK)])

        def ini(k, _):
            macc[pl.ds(k * LL, LL)] = jnp.full((LL,), SENT, jnp.float32)
            return 0
        lax.fori_loop(0, NPAD // LL, ini, 0)

        pltpu.sync_copy(src_h.at[pl.ds(base, EPW)], srcb)
        pltpu.sync_copy(dst_h.at[pl.ds(base, EPW)], dstb)

        def gbody(gg, _):
            s16 = srcb[pl.ds(gg * LL, LL)]
            d16 = dstb[pl.ds(gg * LL, LL)]
            val = plsc.load_gather(wv, [s16])
            cur = plsc.load_gather(macc, [d16])
            need = val > cur

            def cond(carry):
                return carry[0]

            def wbody(carry):
                _, nd = carry
                plsc.store_scatter(macc, [d16], val, mask=nd)
                cur2 = plsc.load_gather(macc, [d16])
                nd2 = val > cur2
                return (jnp.any(nd2), nd2)

            lax.while_loop(cond, wbody, (jnp.any(need), need))
            return 0
        lax.fori_loop(0, EPW // LL, gbody, 0)

        pltpu.sync_copy(macc, shared.at[sid])
        plsc.subcore_barrier()

        def zb(k, _):
            mrg[pl.ds(k * LL, LL)] = jnp.full((LL,), SENT, jnp.float32)
            return 0
        lax.fori_loop(0, NCHK // LL, zb, 0)

        def slot_body(t, _):
            pltpu.sync_copy(shared.at[t, pl.ds(sid * NCHK, NCHK)], tmp)

            def mk(k, _):
                sl = pl.ds(k * LL, LL)
                mrg[sl] = jnp.maximum(mrg[sl], tmp[sl])
                return 0
            lax.fori_loop(0, NCHK // LL, mk, 0)
            return 0
        lax.fori_loop(0, NS, slot_body, 0)
        pltpu.sync_copy(mrg, out_h.at[cid, pl.ds(sid * NCHK, NCHK)])

    if with_prev:
        out_type = jax.ShapeDtypeStruct((NC, NPAD), jnp.float32)
    else:
        out_type = (jax.ShapeDtypeStruct((NC, NPAD), jnp.float32),
                    jax.ShapeDtypeStruct((NPAD,), jnp.float32))
    return functools.partial(
        pl.kernel,
        out_type=out_type,
        mesh=_mesh,
        compiler_params=_SC_PARAMS,
        scratch_types=scratch,
    )(body)


_hop1 = _make_hop(False, 0.5)
_hop2 = _make_hop(True, 0.5)


def _stage_d_body(w1_ref, m1_ref, m2_ref, p_ref, b_ref, pred_ref, mf_ref,
                  o_ref):
    def fx(m):
        return jnp.where(m > -1.0e38, m, 0.0)

    w1 = w1_ref[...]
    w2 = jnp.maximum(w1, 0.5 * fx(jnp.maximum(m1_ref[0], m1_ref[1])))
    w3 = jnp.maximum(w2, 0.25 * fx(jnp.maximum(m2_ref[0], m2_ref[1])))
    resid = (p_ref[3] + p_ref[7]) - b_ref[...]
    loss_pde = jnp.sum(w3 * resid * resid) / NN
    dv = p_ref[2] + p_ref[6]
    loss_cons = jnp.sum(dv * dv) / NN
    pred = pred_ref[...]

    def mmse(mf, t):
        c = jnp.maximum(jnp.sum(mf), 1.0)
        return jnp.sum(mf * (pred - t) ** 2) / c

    loss_bc = mmse(mf_ref[0], 0.1) + mmse(mf_ref[1], 0.0) + mmse(mf_ref[2], 0.0)
    o_ref[0] = loss_pde + 10.0 * loss_bc + loss_cons
    o_ref[1] = loss_pde
    o_ref[2] = loss_bc
    o_ref[3] = loss_cons


_stage_d = pl.pallas_call(
    _stage_d_body,
    out_shape=jax.ShapeDtypeStruct((4,), jnp.float32),
    out_specs=pl.BlockSpec(memory_space=pltpu.MemorySpace.SMEM),
)


def _pad1(x, val=0.0):
    return jnp.concatenate(
        [x, jnp.full((NPAD - NN,), val, x.dtype)])


def kernel(pred, feats, A_row_ptr, A_col_ind, A_vals, A_row_idx, b,
           edge_index, epoch, inlet_mask, outlet_mask, wall_mask):
    f32 = jnp.float32
    i32 = jnp.int32
    src = edge_index[0].astype(i32)
    dst = edge_index[1].astype(i32)
    col = A_col_ind.astype(i32)
    row = A_row_idx.astype(i32)
    vals = A_vals.astype(f32)
    pred_p = _pad1(pred.astype(f32))
    b_p = _pad1(b.astype(f32))

    feats_pk = lax.bitcast_convert_type(
        feats.astype(jnp.bfloat16).reshape(NN, DD // 2, 2), i32)
    partials = _stage_a(src, dst, col, row, vals, pred_p,
                        feats_pk)
    m1p, w1f = _hop1(partials.reshape(NC * 4, NPAD), src, dst)
    m2p = _hop2(w1f, m1p, src, dst)

    sh = (NPAD // 128, 128)
    mf = jnp.stack([
        _pad1(inlet_mask.astype(f32)).reshape(sh),
        _pad1(outlet_mask.astype(f32)).reshape(sh),
        _pad1(wall_mask.astype(f32)).reshape(sh),
    ])
    out4 = _stage_d(
        w1f.reshape(sh),
        m1p.reshape(2, *sh),
        m2p.reshape(2, *sh),
        partials.reshape(8, *sh),
        b_p.reshape(sh),
        pred_p.reshape(sh),
        mf,
    )
    return (out4[0], out4[1], out4[2], out4[3])


# speedup vs baseline: 1.3751x; 1.2273x over previous
import functools

import jax
import jax.numpy as jnp
from jax import lax
from jax.experimental import pallas as pl
from jax.experimental.pallas import tpu as pltpu
from jax.experimental.pallas import tpu_sc as plsc

NN = 10000
EE = 320000
DD = 128
NPAD = 10240
NC = 2
NS = 16
LL = 16
NW = NC * NS
EPW = EE // NW
CH = 80
NCHUNK = EPW // CH
CSR_C = 400
NCSR = EPW // CSR_C
NCHK = NPAD // NS
SENT = -3.0e38

_mesh = plsc.VectorSubcoreMesh(core_axis_name="c", subcore_axis_name="s")
_SC_PARAMS = pltpu.CompilerParams(needs_layout_passes=False)
_SC_PARAMS_A = pltpu.CompilerParams(
    needs_layout_passes=False, use_tc_tiling_on_sc=False)


def _vsqrt(x):
    xi = plsc.bitcast(x, jnp.int32)
    yi = jnp.int32(0x5F3759DF) - (xi >> 1)
    y = plsc.bitcast(yi, jnp.float32)
    for _ in range(4):
        y = y * (1.5 - 0.5 * x * y * y)
    return jnp.where(x <= 0.0, 0.0, x * y)


def _worker_id():
    c = lax.axis_index("c")
    s = lax.axis_index("s")
    return c, s, c * NS + s


@functools.partial(
    pl.kernel,
    out_type=jax.ShapeDtypeStruct((NC, 4, NPAD), jnp.float32),
    mesh=_mesh,
    compiler_params=_SC_PARAMS_A,
    scratch_types=[
        pltpu.VMEM((NPAD,), jnp.float32),
        pltpu.VMEM((NPAD,), jnp.float32),
        pltpu.VMEM((NPAD,), jnp.float32),
        pltpu.VMEM((NPAD,), jnp.float32),
        pltpu.VMEM((NPAD,), jnp.float32),
        pltpu.VMEM((EPW,), jnp.int32),
        pltpu.VMEM((EPW,), jnp.int32),
        pltpu.VMEM((2, CH, DD // 2), jnp.int32),
        pltpu.VMEM((2, CH, DD // 2), jnp.int32),
        pltpu.VMEM((LL * LL,), jnp.float32),
        pltpu.VMEM((2, CSR_C), jnp.int32),
        pltpu.VMEM((2, CSR_C), jnp.int32),
        pltpu.VMEM((2, CSR_C), jnp.float32),
        pltpu.VMEM((NCHK,), jnp.float32),
        pltpu.VMEM((NCHK,), jnp.float32),
        pltpu.VMEM_SHARED((NS, 2, NPAD), jnp.float32),
        pltpu.SemaphoreType.DMA((2,)),
        pltpu.SemaphoreType.DMA((2,)),
        pltpu.SemaphoreType.DMA((2,)),
    ],
)
def _stage_a(src_h, dst_h, col_h, row_h, val_h, pred_h, feats_h, out_h,
             predv, ngacc, cntacc, divacc, rracc, srcb, dstb, rs, rd,
             tbuf, colb, rowb, valb, tmp, mrg, shared, sems, semd, semc):
    cid, sid, wid = _worker_id()
    base = wid * EPW

    pltpu.sync_copy(pred_h, predv)
    pltpu.sync_copy(src_h.at[pl.ds(base, EPW)], srcb)
    pltpu.sync_copy(dst_h.at[pl.ds(base, EPW)], dstb)

    def zbody(k, _):
        z = jnp.zeros((LL,), jnp.float32)
        ngacc[pl.ds(k * LL, LL)] = z
        cntacc[pl.ds(k * LL, LL)] = z
        divacc[pl.ds(k * LL, LL)] = z
        rracc[pl.ds(k * LL, LL)] = z
        return 0
    lax.fori_loop(0, NPAD // LL, zbody, 0)

    ones16 = jnp.ones((LL,), jnp.float32)

    def launch(i, slot):
        pltpu.async_copy(
            feats_h.at[srcb.at[pl.ds(i * CH, CH)]], rs.at[slot],
            sems.at[slot])
        pltpu.async_copy(
            feats_h.at[dstb.at[pl.ds(i * CH, CH)]], rd.at[slot],
            semd.at[slot])

    def do_chunk(i, slot):
        pltpu.make_async_copy(
            feats_h.at[srcb.at[pl.ds(i * CH, CH)]], rs.at[slot],
            sems.at[slot]).wait()
        pltpu.make_async_copy(
            feats_h.at[dstb.at[pl.ds(i * CH, CH)]], rd.at[slot],
            semd.at[slot]).wait()

        @pl.when(i + 1 < NCHUNK)
        def _():
            launch(i + 1, 1 - slot)

        for g in range(CH // LL):
            s16 = srcb[pl.ds(i * CH + g * LL, LL)]
            d16 = dstb[pl.ds(i * CH + g * LL, LL)]

            himask = jnp.int32(-65536)

            def erow(el, _):
                e = g * LL + el
                acc = jnp.zeros((LL,), jnp.float32)
                for k in range(DD // (2 * LL)):
                    a = rs[slot, e, pl.ds(k * LL, LL)]
                    b2 = rd[slot, e, pl.ds(k * LL, LL)]
                    dlo = (plsc.bitcast(a << 16, jnp.float32)
                           - plsc.bitcast(b2 << 16, jnp.float32))
                    dhi = (plsc.bitcast(a & himask, jnp.float32)
                           - plsc.bitcast(b2 & himask, jnp.float32))
                    acc = acc + dlo * dlo + dhi * dhi
                tbuf[pl.ds(el * LL, LL)] = acc
                return 0
            lax.fori_loop(0, LL, erow, 0)

            d2 = jnp.zeros((LL,), jnp.float32)
            rowi = lax.iota(jnp.int32, LL) * LL
            for c in range(LL):
                d2 = d2 + plsc.load_gather(tbuf, [rowi + c])

            ps = plsc.load_gather(predv, [s16])
            pd = plsc.load_gather(predv, [d16])
            pdiff = jnp.abs(pd - ps)
            flux = ps - pd
            fd = _vsqrt(d2)
            gval = pdiff + 0.1 * fd
            plsc.addupdate_scatter(ngacc, [d16], gval)
            plsc.addupdate_scatter(cntacc, [d16], ones16)
            plsc.addupdate_scatter(divacc, [d16], flux)

    launch(0, 0)

    def pair_body(j, _):
        do_chunk(2 * j, 0)
        do_chunk(2 * j + 1, 1)
        return 0
    lax.fori_loop(0, (NCHUNK - 1) // 2, pair_body, 0)
    do_chunk(NCHUNK - 1, 0)

    def csr_launch(t, slot):
        off = base + t * CSR_C
        pltpu.async_copy(col_h.at[pl.ds(off, CSR_C)], colb.at[slot],
                         semc.at[slot])
        pltpu.async_copy(row_h.at[pl.ds(off, CSR_C)], rowb.at[slot],
                         semc.at[slot])
        pltpu.async_copy(val_h.at[pl.ds(off, CSR_C)], valb.at[slot],
                         semc.at[slot])

    def csr_chunk(t, slot):
        off = base + t * CSR_C
        pltpu.make_async_copy(col_h.at[pl.ds(off, CSR_C)], colb.at[slot],
                              semc.at[slot]).wait()
        pltpu.make_async_copy(row_h.at[pl.ds(off, CSR_C)], rowb.at[slot],
                              semc.at[slot]).wait()
        pltpu.make_async_copy(val_h.at[pl.ds(off, CSR_C)], valb.at[slot],
                              semc.at[slot]).wait()

        @pl.when(t + 1 < NCSR)
        def _():
            csr_launch(t + 1, 1 - slot)

        def gbody(gg, _):
            c16 = colb[slot, pl.ds(gg * LL, LL)]
            r16 = rowb[slot, pl.ds(gg * LL, LL)]
            v16 = valb[slot, pl.ds(gg * LL, LL)]
            pv = plsc.load_gather(predv, [c16])
            plsc.addupdate_scatter(rracc, [r16], v16 * pv)
            return 0
        lax.fori_loop(0, CSR_C // LL, gbody, 0)

    csr_launch(0, 0)

    def csr_pair(j, _):
        csr_chunk(2 * j, 0)
        csr_chunk(2 * j + 1, 1)
        return 0
    lax.fori_loop(0, (NCSR - 1) // 2, csr_pair, 0)
    csr_chunk(NCSR - 1, 0)

    for half, pair in enumerate(((ngacc, cntacc), (divacc, rracc))):
        if half:
            plsc.subcore_barrier()
        pltpu.sync_copy(pair[0], shared.at[sid, 0])
        pltpu.sync_copy(pair[1], shared.at[sid, 1])
        plsc.subcore_barrier()

        for r in range(2):
            def zb(k, _):
                mrg[pl.ds(k * LL, LL)] = jnp.zeros((LL,), jnp.float32)
                return 0
            lax.fori_loop(0, NCHK // LL, zb, 0)

            def slot_body(t, _):
                pltpu.sync_copy(
                    shared.at[t, r, pl.ds(sid * NCHK, NCHK)], tmp)

                def addk(k, _):
                    sl = pl.ds(k * LL, LL)
                    mrg[sl] = mrg[sl] + tmp[sl]
                    return 0
                lax.fori_loop(0, NCHK // LL, addk, 0)
                return 0
            lax.fori_loop(0, NS, slot_body, 0)
            pltpu.sync_copy(
                mrg, out_h.at[cid, half * 2 + r, pl.ds(sid * NCHK, NCHK)])


def _make_hop(with_prev, decay):
    scratch = [
        pltpu.VMEM((NPAD,), jnp.float32),
        pltpu.VMEM((NPAD,), jnp.float32),
        pltpu.VMEM((EPW,), jnp.int32),
        pltpu.VMEM((EPW,), jnp.int32),
        pltpu.VMEM((NCHK,), jnp.float32),
        pltpu.VMEM((NCHK,), jnp.float32),
        pltpu.VMEM((NPAD,), jnp.float32),
        pltpu.VMEM((NPAD,), jnp.float32),
        pltpu.VMEM_SHARED((NS, NPAD), jnp.float32),
    ]

    def body(*refs):
        if with_prev:
            (w_h, mp_h, src_h, dst_h, out_h,
             wv, macc, srcb, dstb, tmp, mrg, mp0, mp1, shared) = refs
        else:
            (p_h, src_h, dst_h, out_h, w_out,
             wv, macc, srcb, dstb, tmp, mrg, mp0, mp1, shared) = refs
        cid, sid, wid = _worker_id()
        base = wid * EPW

        if with_prev:
            pltpu.sync_copy(w_h, wv)
            pltpu.sync_copy(mp_h.at[0], mp0)
            pltpu.sync_copy(mp_h.at[1], mp1)

            def upd(k, _):
                sl = pl.ds(k * LL, LL)
                m = jnp.maximum(mp0[sl], mp1[sl])
                fx = jnp.where(m > -1.0e38, m, 0.0)
                wv[sl] = jnp.maximum(wv[sl], decay * fx)
                return 0
            lax.fori_loop(0, NPAD // LL, upd, 0)
        else:
            pltpu.sync_copy(p_h.at[0], wv)
            pltpu.sync_copy(p_h.at[4], mp0)
            pltpu.sync_copy(p_h.at[1], mp1)

            def u1(k, _):
                sl = pl.ds(k * LL, LL)
                wv[sl] = wv[sl] + mp0[sl]
                return 0
            lax.fori_loop(0, NPAD // LL, u1, 0)
            pltpu.sync_copy(p_h.at[5], mp0)

            def u2(k, _):
                sl = pl.ds(k * LL, LL)
                wv[sl] = wv[sl] / (mp0[sl] + mp1[sl] + 1.0)
                return 0
            lax.fori_loop(0, NPAD // LL, u2, 0)

            def u3(k, acc):
                return acc + wv[pl.ds(k * LL, LL)]
            vec = lax.fori_loop(0, NPAD // LL, u3,
                                jnp.zeros((LL,), jnp.float32))
            mean = jnp.sum(vec) * jnp.float32(1.0 / NN)

            def u4(k, _):
                sl = pl.ds(k * LL, LL)
                wv[sl] = 1.0 + wv[sl] / (mean + 1e-8)
                return 0
            lax.fori_loop(0, NPAD // LL, u4, 0)

            @pl.when(cid == 0)
            def _():
                pltpu.sync_copy(wv.at[pl.ds(sid * NCHK, NCHK)],
                                w_out.at[pl.ds(sid * NCHK, NCHK)])

        def ini(k, _):
            macc[pl.ds(k * LL, LL)] = jnp.full((LL,), SENT, jnp.float32)
            return 0
        lax.fori_loop(0, NPAD // LL, ini, 0)

        pltpu.sync_copy(src_h.at[pl.ds(base, EPW)], srcb)
        pltpu.sync_copy(dst_h.at[pl.ds(base, EPW)], dstb)

        def gbody(gg, _):
            s16 = srcb[pl.ds(gg * LL, LL)]
            d16 = dstb[pl.ds(gg * LL, LL)]
            val = plsc.load_gather(wv, [s16])
            cur = plsc.load_gather(macc, [d16])
            need = val > cur

            def cond(carry):
                return carry[0]

            def wbody(carry):
                _, nd = carry
                plsc.store_scatter(macc, [d16], val, mask=nd)
                cur2 = plsc.load_gather(macc, [d16])
                nd2 = val > cur2
                return (jnp.any(nd2), nd2)

            lax.while_loop(cond, wbody, (jnp.any(need), need))
            return 0
        lax.fori_loop(0, EPW // LL, gbody, 0)

        pltpu.sync_copy(macc, shared.at[sid])
        plsc.subcore_barrier()

        def zb(k, _):
            mrg[pl.ds(k * LL, LL)] = jnp.full((LL,), SENT, jnp.float32)
            return 0
        lax.fori_loop(0, NCHK // LL, zb, 0)

        def slot_body(t, _):
            pltpu.sync_copy(shared.at[t, pl.ds(sid * NCHK, NCHK)], tmp)

            def mk(k, _):
                sl = pl.ds(k * LL, LL)
                mrg[sl] = jnp.maximum(mrg[sl], tmp[sl])
                return 0
            lax.fori_loop(0, NCHK // LL, mk, 0)
            return 0
        lax.fori_loop(0, NS, slot_body, 0)
        pltpu.sync_copy(mrg, out_h.at[cid, pl.ds(sid * NCHK, NCHK)])

    if with_prev:
        out_type = jax.ShapeDtypeStruct((NC, NPAD), jnp.float32)
    else:
        out_type = (jax.ShapeDtypeStruct((NC, NPAD), jnp.float32),
                    jax.ShapeDtypeStruct((NPAD,), jnp.float32))
    return functools.partial(
        pl.kernel,
        out_type=out_type,
        mesh=_mesh,
        compiler_params=_SC_PARAMS,
        scratch_types=scratch,
    )(body)


_hop1 = _make_hop(False, 0.5)
_hop2 = _make_hop(True, 0.5)


def _stage_d_body(w1_ref, m1_ref, m2_ref, p_ref, b_ref, pred_ref, mf_ref,
                  o_ref):
    def fx(m):
        return jnp.where(m > -1.0e38, m, 0.0)

    w1 = w1_ref[...]
    w2 = jnp.maximum(w1, 0.5 * fx(jnp.maximum(m1_ref[0], m1_ref[1])))
    w3 = jnp.maximum(w2, 0.25 * fx(jnp.maximum(m2_ref[0], m2_ref[1])))
    resid = (p_ref[3] + p_ref[7]) - b_ref[...]
    loss_pde = jnp.sum(w3 * resid * resid) / NN
    dv = p_ref[2] + p_ref[6]
    loss_cons = jnp.sum(dv * dv) / NN
    pred = pred_ref[...]

    def mmse(mf, t):
        c = jnp.maximum(jnp.sum(mf), 1.0)
        return jnp.sum(mf * (pred - t) ** 2) / c

    loss_bc = mmse(mf_ref[0], 0.1) + mmse(mf_ref[1], 0.0) + mmse(mf_ref[2], 0.0)
    o_ref[0] = loss_pde + 10.0 * loss_bc + loss_cons
    o_ref[1] = loss_pde
    o_ref[2] = loss_bc
    o_ref[3] = loss_cons


_stage_d = pl.pallas_call(
    _stage_d_body,
    out_shape=jax.ShapeDtypeStruct((4,), jnp.float32),
    out_specs=pl.BlockSpec(memory_space=pltpu.MemorySpace.SMEM),
)


def _pad1(x, val=0.0):
    return jnp.concatenate(
        [x, jnp.full((NPAD - NN,), val, x.dtype)])


def kernel(pred, feats, A_row_ptr, A_col_ind, A_vals, A_row_idx, b,
           edge_index, epoch, inlet_mask, outlet_mask, wall_mask):
    f32 = jnp.float32
    i32 = jnp.int32
    src = edge_index[0].astype(i32)
    dst = edge_index[1].astype(i32)
    col = A_col_ind.astype(i32)
    row = A_row_idx.astype(i32)
    vals = A_vals.astype(f32)
    pred_p = _pad1(pred.astype(f32))
    b_p = _pad1(b.astype(f32))

    feats_pk = lax.bitcast_convert_type(
        feats.astype(jnp.bfloat16).reshape(NN, DD // 2, 2), i32)
    partials = _stage_a(src, dst, col, row, vals, pred_p,
                        feats_pk)
    m1p, w1f = _hop1(partials.reshape(NC * 4, NPAD), src, dst)
    m2p = _hop2(w1f, m1p, src, dst)

    sh = (NPAD // 128, 128)
    mf = jnp.stack([
        _pad1(inlet_mask.astype(f32)).reshape(sh),
        _pad1(outlet_mask.astype(f32)).reshape(sh),
        _pad1(wall_mask.astype(f32)).reshape(sh),
    ])
    out4 = _stage_d(
        w1f.reshape(sh),
        m1p.reshape(2, *sh),
        m2p.reshape(2, *sh),
        partials.reshape(8, *sh),
        b_p.reshape(sh),
        pred_p.reshape(sh),
        mf,
    )
    return (out4[0], out4[1], out4[2], out4[3])


# speedup vs baseline: 1.5141x; 1.1011x over previous
import functools

import jax
import jax.numpy as jnp
from jax import lax
from jax.experimental import pallas as pl
from jax.experimental.pallas import tpu as pltpu
from jax.experimental.pallas import tpu_sc as plsc

NN = 10000
EE = 320000
DD = 128
NPAD = 10240
NC = 2
NS = 16
LL = 16
NW = NC * NS
EPW = EE // NW
CH = 80
NCHUNK = EPW // CH
CSR_C = 400
NCSR = EPW // CSR_C
NCHK = NPAD // NS
SENT = -3.0e38

_mesh = plsc.VectorSubcoreMesh(core_axis_name="c", subcore_axis_name="s")
_SC_PARAMS = pltpu.CompilerParams(needs_layout_passes=False)
_SC_PARAMS_A = pltpu.CompilerParams(
    needs_layout_passes=False, use_tc_tiling_on_sc=False)


def _vsqrt(x):
    xi = plsc.bitcast(x, jnp.int32)
    yi = jnp.int32(0x5F3759DF) - (xi >> 1)
    y = plsc.bitcast(yi, jnp.float32)
    for _ in range(4):
        y = y * (1.5 - 0.5 * x * y * y)
    return jnp.where(x <= 0.0, 0.0, x * y)


def _worker_id():
    c = lax.axis_index("c")
    s = lax.axis_index("s")
    return c, s, c * NS + s


@functools.partial(
    pl.kernel,
    out_type=jax.ShapeDtypeStruct((NC, 4, NPAD), jnp.float32),
    mesh=_mesh,
    compiler_params=_SC_PARAMS_A,
    scratch_types=[
        pltpu.VMEM((NPAD,), jnp.float32),
        pltpu.VMEM((NPAD,), jnp.float32),
        pltpu.VMEM((NPAD,), jnp.float32),
        pltpu.VMEM((NPAD,), jnp.float32),
        pltpu.VMEM((NPAD,), jnp.float32),
        pltpu.VMEM((EPW,), jnp.int32),
        pltpu.VMEM((EPW,), jnp.int32),
        pltpu.VMEM((2, CH, DD), jnp.bfloat16),
        pltpu.VMEM((2, CH, DD), jnp.bfloat16),
        pltpu.VMEM((LL * LL,), jnp.float32),
        pltpu.VMEM((2, CSR_C), jnp.int32),
        pltpu.VMEM((2, CSR_C), jnp.int32),
        pltpu.VMEM((2, CSR_C), jnp.float32),
        pltpu.VMEM((NCHK,), jnp.float32),
        pltpu.VMEM((NCHK,), jnp.float32),
        pltpu.VMEM_SHARED((NS, 2, NPAD), jnp.float32),
        pltpu.SemaphoreType.DMA((2,)),
        pltpu.SemaphoreType.DMA((2,)),
        pltpu.SemaphoreType.DMA((2,)),
    ],
)
def _stage_a(src_h, dst_h, col_h, row_h, val_h, pred_h, feats_h, out_h,
             predv, ngacc, cntacc, divacc, rracc, srcb, dstb, rs, rd,
             tbuf, colb, rowb, valb, tmp, mrg, shared, sems, semd, semc):
    cid, sid, wid = _worker_id()
    base = wid * EPW

    pltpu.sync_copy(pred_h, predv)
    pltpu.sync_copy(src_h.at[pl.ds(base, EPW)], srcb)
    pltpu.sync_copy(dst_h.at[pl.ds(base, EPW)], dstb)

    def zbody(k, _):
        z = jnp.zeros((LL,), jnp.float32)
        ngacc[pl.ds(k * LL, LL)] = z
        cntacc[pl.ds(k * LL, LL)] = z
        divacc[pl.ds(k * LL, LL)] = z
        rracc[pl.ds(k * LL, LL)] = z
        return 0
    lax.fori_loop(0, NPAD // LL, zbody, 0)

    ones16 = jnp.ones((LL,), jnp.float32)

    def launch(i, slot):
        pltpu.async_copy(
            feats_h.at[srcb.at[pl.ds(i * CH, CH)]], rs.at[slot],
            sems.at[slot])
        pltpu.async_copy(
            feats_h.at[dstb.at[pl.ds(i * CH, CH)]], rd.at[slot],
            semd.at[slot])

    def do_chunk(i, slot):
        pltpu.make_async_copy(
            feats_h.at[srcb.at[pl.ds(i * CH, CH)]], rs.at[slot],
            sems.at[slot]).wait()
        pltpu.make_async_copy(
            feats_h.at[dstb.at[pl.ds(i * CH, CH)]], rd.at[slot],
            semd.at[slot]).wait()

        @pl.when(i + 1 < NCHUNK)
        def _():
            launch(i + 1, 1 - slot)

        for g in range(CH // LL):
            s16 = srcb[pl.ds(i * CH + g * LL, LL)]
            d16 = dstb[pl.ds(i * CH + g * LL, LL)]

            def erow(el, _):
                e = g * LL + el
                acc = jnp.zeros((LL,), jnp.float32)
                for k in range(DD // (2 * LL)):
                    a = rs[slot, e, pl.ds(k * 2 * LL, 2 * LL)]
                    b2 = rd[slot, e, pl.ds(k * 2 * LL, 2 * LL)]
                    d32 = a - b2
                    dlo, dhi = plsc.unpack(
                        d32, format=plsc.PackFormat.INTERLEAVED)
                    acc = acc + dlo * dlo + dhi * dhi
                tbuf[pl.ds(el * LL, LL)] = acc
                return 0
            lax.fori_loop(0, LL, erow, 0)

            d2 = jnp.zeros((LL,), jnp.float32)
            rowi = lax.iota(jnp.int32, LL) * LL
            for c in range(LL):
                d2 = d2 + plsc.load_gather(tbuf, [rowi + c])

            ps = plsc.load_gather(predv, [s16])
            pd = plsc.load_gather(predv, [d16])
            pdiff = jnp.abs(pd - ps)
            flux = ps - pd
            fd = _vsqrt(d2)
            gval = pdiff + 0.1 * fd
            plsc.addupdate_scatter(ngacc, [d16], gval)
            plsc.addupdate_scatter(cntacc, [d16], ones16)
            plsc.addupdate_scatter(divacc, [d16], flux)

    launch(0, 0)

    def pair_body(j, _):
        do_chunk(2 * j, 0)
        do_chunk(2 * j + 1, 1)
        return 0
    lax.fori_loop(0, (NCHUNK - 1) // 2, pair_body, 0)
    do_chunk(NCHUNK - 1, 0)

    def csr_launch(t, slot):
        off = base + t * CSR_C
        pltpu.async_copy(col_h.at[pl.ds(off, CSR_C)], colb.at[slot],
                         semc.at[slot])
        pltpu.async_copy(row_h.at[pl.ds(off, CSR_C)], rowb.at[slot],
                         semc.at[slot])
        pltpu.async_copy(val_h.at[pl.ds(off, CSR_C)], valb.at[slot],
                         semc.at[slot])

    def csr_chunk(t, slot):
        off = base + t * CSR_C
        pltpu.make_async_copy(col_h.at[pl.ds(off, CSR_C)], colb.at[slot],
                              semc.at[slot]).wait()
        pltpu.make_async_copy(row_h.at[pl.ds(off, CSR_C)], rowb.at[slot],
                              semc.at[slot]).wait()
        pltpu.make_async_copy(val_h.at[pl.ds(off, CSR_C)], valb.at[slot],
                              semc.at[slot]).wait()

        @pl.when(t + 1 < NCSR)
        def _():
            csr_launch(t + 1, 1 - slot)

        def gbody(gg, _):
            c16 = colb[slot, pl.ds(gg * LL, LL)]
            r16 = rowb[slot, pl.ds(gg * LL, LL)]
            v16 = valb[slot, pl.ds(gg * LL, LL)]
            pv = plsc.load_gather(predv, [c16])
            plsc.addupdate_scatter(rracc, [r16], v16 * pv)
            return 0
        lax.fori_loop(0, CSR_C // LL, gbody, 0)

    csr_launch(0, 0)

    def csr_pair(j, _):
        csr_chunk(2 * j, 0)
        csr_chunk(2 * j + 1, 1)
        return 0
    lax.fori_loop(0, (NCSR - 1) // 2, csr_pair, 0)
    csr_chunk(NCSR - 1, 0)

    for half, pair in enumerate(((ngacc, cntacc), (divacc, rracc))):
        if half:
            plsc.subcore_barrier()
        pltpu.sync_copy(pair[0], shared.at[sid, 0])
        pltpu.sync_copy(pair[1], shared.at[sid, 1])
        plsc.subcore_barrier()

        for r in range(2):
            def zb(k, _):
                mrg[pl.ds(k * LL, LL)] = jnp.zeros((LL,), jnp.float32)
                return 0
            lax.fori_loop(0, NCHK // LL, zb, 0)

            def slot_body(t, _):
                pltpu.sync_copy(
                    shared.at[t, r, pl.ds(sid * NCHK, NCHK)], tmp)

                def addk(k, _):
                    sl = pl.ds(k * LL, LL)
                    mrg[sl] = mrg[sl] + tmp[sl]
                    return 0
                lax.fori_loop(0, NCHK // LL, addk, 0)
                return 0
            lax.fori_loop(0, NS, slot_body, 0)
            pltpu.sync_copy(
                mrg, out_h.at[cid, half * 2 + r, pl.ds(sid * NCHK, NCHK)])


def _make_hop(with_prev, decay):
    scratch = [
        pltpu.VMEM((NPAD,), jnp.float32),
        pltpu.VMEM((NPAD,), jnp.float32),
        pltpu.VMEM((EPW,), jnp.int32),
        pltpu.VMEM((EPW,), jnp.int32),
        pltpu.VMEM((NCHK,), jnp.float32),
        pltpu.VMEM((NCHK,), jnp.float32),
        pltpu.VMEM((NPAD,), jnp.float32),
        pltpu.VMEM((NPAD,), jnp.float32),
        pltpu.VMEM_SHARED((NS, NPAD), jnp.float32),
    ]

    def body(*refs):
        if with_prev:
            (w_h, mp_h, src_h, dst_h, out_h,
             wv, macc, srcb, dstb, tmp, mrg, mp0, mp1, shared) = refs
        else:
            (w_h, src_h, dst_h, out_h,
             wv, macc, srcb, dstb, tmp, mrg, mp0, mp1, shared) = refs
        cid, sid, wid = _worker_id()
        base = wid * EPW

        pltpu.sync_copy(w_h, wv)
        if with_prev:
            pltpu.sync_copy(mp_h.at[0], mp0)
            pltpu.sync_copy(mp_h.at[1], mp1)

            def upd(k, _):
                sl = pl.ds(k * LL, LL)
                m = jnp.maximum(mp0[sl], mp1[sl])
                fx = jnp.where(m > -1.0e38, m, 0.0)
                wv[sl] = jnp.maximum(wv[sl], decay * fx)
                return 0
            lax.fori_loop(0, NPAD // LL, upd, 0)

        def ini(k, _):
            macc[pl.ds(k * LL, LL)] = jnp.full((LL,), SENT, jnp.float32)
            return 0
        lax.fori_loop(0, NPAD // LL, ini, 0)

        pltpu.sync_copy(src_h.at[pl.ds(base, EPW)], srcb)
        pltpu.sync_copy(dst_h.at[pl.ds(base, EPW)], dstb)

        def gbody(gg, _):
            s16 = srcb[pl.ds(gg * LL, LL)]
            d16 = dstb[pl.ds(gg * LL, LL)]
            val = plsc.load_gather(wv, [s16])
            cur = plsc.load_gather(macc, [d16])
            need = val > cur

            def cond(carry):
                return carry[0]

            def wbody(carry):
                _, nd = carry
                plsc.store_scatter(macc, [d16], val, mask=nd)
                cur2 = plsc.load_gather(macc, [d16])
                nd2 = val > cur2
                return (jnp.any(nd2), nd2)

            lax.while_loop(cond, wbody, (jnp.any(need), need))
            return 0
        lax.fori_loop(0, EPW // LL, gbody, 0)

        pltpu.sync_copy(macc, shared.at[sid])
        plsc.subcore_barrier()

        def zb(k, _):
            mrg[pl.ds(k * LL, LL)] = jnp.full((LL,), SENT, jnp.float32)
            return 0
        lax.fori_loop(0, NCHK // LL, zb, 0)

        def slot_body(t, _):
            pltpu.sync_copy(shared.at[t, pl.ds(sid * NCHK, NCHK)], tmp)

            def mk(k, _):
                sl = pl.ds(k * LL, LL)
                mrg[sl] = jnp.maximum(mrg[sl], tmp[sl])
                return 0
            lax.fori_loop(0, NCHK // LL, mk, 0)
            return 0
        lax.fori_loop(0, NS, slot_body, 0)
        pltpu.sync_copy(mrg, out_h.at[cid, pl.ds(sid * NCHK, NCHK)])

    return functools.partial(
        pl.kernel,
        out_type=jax.ShapeDtypeStruct((NC, NPAD), jnp.float32),
        mesh=_mesh,
        compiler_params=_SC_PARAMS,
        scratch_types=scratch,
    )(body)


_hop1 = _make_hop(False, 0.5)
_hop2 = _make_hop(True, 0.5)


def _stage_b_body(p_ref, o_ref):
    ng = p_ref[0] + p_ref[4]
    cnt = p_ref[1] + p_ref[5]
    g = ng / (cnt + 1.0)
    mean = jnp.sum(g) / NN
    o_ref[...] = 1.0 + g / (mean + 1e-8)


_stage_b = pl.pallas_call(
    _stage_b_body,
    out_shape=jax.ShapeDtypeStruct((NPAD // 128, 128), jnp.float32),
)


def _stage_d_body(w1_ref, m1_ref, m2_ref, p_ref, b_ref, pred_ref, mf_ref,
                  o_ref):
    def fx(m):
        return jnp.where(m > -1.0e38, m, 0.0)

    w1 = w1_ref[...]
    w2 = jnp.maximum(w1, 0.5 * fx(jnp.maximum(m1_ref[0], m1_ref[1])))
    w3 = jnp.maximum(w2, 0.25 * fx(jnp.maximum(m2_ref[0], m2_ref[1])))
    resid = (p_ref[3] + p_ref[7]) - b_ref[...]
    loss_pde = jnp.sum(w3 * resid * resid) / NN
    dv = p_ref[2] + p_ref[6]
    loss_cons = jnp.sum(dv * dv) / NN
    pred = pred_ref[...]

    def mmse(mf, t):
        c = jnp.maximum(jnp.sum(mf), 1.0)
        return jnp.sum(mf * (pred - t) ** 2) / c

    loss_bc = mmse(mf_ref[0], 0.1) + mmse(mf_ref[1], 0.0) + mmse(mf_ref[2], 0.0)
    o_ref[0] = loss_pde + 10.0 * loss_bc + loss_cons
    o_ref[1] = loss_pde
    o_ref[2] = loss_bc
    o_ref[3] = loss_cons


_stage_d = pl.pallas_call(
    _stage_d_body,
    out_shape=jax.ShapeDtypeStruct((4,), jnp.float32),
    out_specs=pl.BlockSpec(memory_space=pltpu.MemorySpace.SMEM),
)


def _pad1(x, val=0.0):
    return jnp.concatenate(
        [x, jnp.full((NPAD - NN,), val, x.dtype)])


def kernel(pred, feats, A_row_ptr, A_col_ind, A_vals, A_row_idx, b,
           edge_index, epoch, inlet_mask, outlet_mask, wall_mask):
    f32 = jnp.float32
    i32 = jnp.int32
    src = edge_index[0].astype(i32)
    dst = edge_index[1].astype(i32)
    col = A_col_ind.astype(i32)
    row = A_row_idx.astype(i32)
    vals = A_vals.astype(f32)
    pred_p = _pad1(pred.astype(f32))
    b_p = _pad1(b.astype(f32))

    partials = _stage_a(src, dst, col, row, vals, pred_p,
                        feats.astype(jnp.bfloat16))
    w1 = _stage_b(partials.reshape(8, NPAD // 128, 128))
    w1f = w1.reshape(NPAD)
    m1p = _hop1(w1f, src, dst)
    m2p = _hop2(w1f, m1p, src, dst)

    sh = (NPAD // 128, 128)
    mf = jnp.stack([
        _pad1(inlet_mask.astype(f32)).reshape(sh),
        _pad1(outlet_mask.astype(f32)).reshape(sh),
        _pad1(wall_mask.astype(f32)).reshape(sh),
    ])
    out4 = _stage_d(
        w1f.reshape(sh),
        m1p.reshape(2, *sh),
        m2p.reshape(2, *sh),
        partials.reshape(8, *sh),
        b_p.reshape(sh),
        pred_p.reshape(sh),
        mf,
    )
    return (out4[0], out4[1], out4[2], out4[3])
